# baseline (speedup 1.0000x reference)
import jax
import jax.numpy as jnp
from jax import lax
from jax.experimental import pallas as pl
from jax.experimental.pallas import tpu as pltpu
from jax.experimental.pallas import tpu_sc as plsc
import functools

B = 4096
S = 200
D = 32

NC = 2
NS = 16
NW = NC * NS

BT = B // 128
DT = D // 8
CP = 4
NCHUNK = S // CP
CROWS = CP * 128

_mesh = plsc.VectorSubcoreMesh(core_axis_name="c", subcore_axis_name="s")


@functools.partial(
    pl.kernel,
    mesh=_mesh,
    compiler_params=pltpu.CompilerParams(use_tc_tiling_on_sc=False,
                                         needs_layout_passes=False),
    out_type=jax.ShapeDtypeStruct((S, DT, BT, 8, 128), jnp.float32),
    scratch_types=[
        pltpu.VMEM((CP, 128), jnp.int32),
        pltpu.VMEM((CP, 128), jnp.int32),
        pltpu.VMEM((CROWS, D), jnp.float32),
        pltpu.VMEM((CROWS, D), jnp.float32),
        pltpu.VMEM((CP, DT, 8, 128), jnp.float32),
        pltpu.VMEM((CP, DT, 8, 128), jnp.float32),
        pltpu.VMEM((S, D), jnp.float32),
        pltpu.SemaphoreType.DMA,
        pltpu.SemaphoreType.DMA,
        pltpu.SemaphoreType.DMA,
        pltpu.SemaphoreType.DMA,
    ],
)
def _embed_sc(xt_hbm, tok_hbm, pos_hbm, out_hbm,
              idx0, idx1, rows0, rows1, tr0, tr1, pos_v, g0, g1, o0, o1):
    wid = lax.axis_index("s") * NC + lax.axis_index("c")
    idx = (idx0, idx1)
    rows = (rows0, rows1)
    tr = (tr0, tr1)
    gs = (g0, g1)
    os_ = (o0, o1)

    pltpu.sync_copy(pos_hbm, pos_v)

    def prefetch(c, b):
        s0 = c * CP
        pltpu.sync_copy(
            xt_hbm.at[pl.ds(s0, CP), pl.ds(wid * 128, 128)], idx[b])
        for si in range(CP):
            pltpu.async_copy(tok_hbm.at[idx[b].at[si]],
                             rows[b].at[pl.ds(si * 128, 128)], gs[b])

    prefetch(0, 0)

    def outer(c2, _):
        for b in range(2):
            c = c2 * 2 + b
            nb = 1 - b

            @pl.when(c < NCHUNK - 1)
            def _():
                prefetch(c + 1, nb)

            pltpu.make_async_copy(
                tok_hbm.at[pl.ds(0, CROWS)], rows[b], gs[b]).wait()

            @pl.when(c >= 2)
            def _():
                pltpu.make_async_copy(
                    tr[b], out_hbm.at[pl.ds(0, CP), :, 0], os_[b]).wait()

            s0 = c * CP

            def tp_body(dl, _):
                iota16 = lax.iota(jnp.int32, 16)
                for si in range(CP):
                    s_vec = jnp.broadcast_to(s0 + si, (16,))
                    for dt in range(DT):
                        d = dt * 8 + dl
                        d_vec = jnp.broadcast_to(d, (16,))
                        ps = plsc.load_gather(pos_v, [s_vec, d_vec])
                        for g in range(8):
                            r_vec = iota16 + (si * 128 + g * 16)
                            v = plsc.load_gather(rows[b], [r_vec, d_vec])
                            tr[b][si, dt, dl, pl.ds(g * 16, 16)] = v + ps
                return 0

            lax.fori_loop(0, 8, tp_body, 0)
            pltpu.async_copy(
                tr[b], out_hbm.at[pl.ds(s0, CP), :, wid], os_[b])
        return 0

    lax.fori_loop(0, NCHUNK // 2, outer, 0)
    pltpu.make_async_copy(tr[0], out_hbm.at[pl.ds(0, CP), :, 0], os_[0]).wait()
    pltpu.make_async_copy(tr[1], out_hbm.at[pl.ds(0, CP), :, 0], os_[1]).wait()


def kernel(x, token_table, pos_table):
    xt = x.astype(jnp.int32).T
    out5 = _embed_sc(xt, token_table, pos_table)
    return out5.transpose(2, 4, 0, 1, 3).reshape(B, S, D)
